# baseline (speedup 1.0000x reference)
import functools

import jax
import jax.numpy as jnp
from jax.experimental import pallas as pl
from jax.experimental.pallas import tpu as pltpu

_K = 1024
_D = 64
_COMMIT = 0.25


def _vq_body(x_ref, e_ref, et_ref, q_ref, i_ref, loss_ref, esq_ref):
    b = pl.program_id(0)
    tb = pl.program_id(1)
    x = x_ref[0]
    emb = e_ref[...]
    embt = et_ref[...]
    tblk = x.shape[1]

    @pl.when((b == 0) & (tb == 0))
    def _init():
        esq_ref[...] = jnp.sum(emb * emb, axis=1)[None, :]
        loss_ref[...] = jnp.zeros((1, 1), jnp.float32)

    xt = jnp.transpose(x)
    x_sq = jnp.sum(xt * xt, axis=1, keepdims=True)
    e_sq = esq_ref[0]
    xe = jax.lax.dot_general(
        xt, embt, (((1,), (0,)), ((), ())),
        preferred_element_type=jnp.float32)
    dist = x_sq + e_sq[None, :] - 2.0 * xe

    dist_t = jnp.transpose(dist)
    minv = jnp.min(dist_t, axis=0, keepdims=True)
    kiota = jax.lax.broadcasted_iota(jnp.int32, (_K, tblk), 0)
    sel = jnp.where(dist_t == minv, kiota, _K)
    idx_row = jnp.min(sel, axis=0, keepdims=True)

    onehot_t = (kiota == idx_row).astype(jnp.float32)
    quant = jax.lax.dot_general(
        embt, onehot_t, (((1,), (0,)), ((), ())),
        preferred_element_type=jnp.float32)

    q_ref[0] = x + (quant - x)
    i_ref[0, 0, 0] = idx_row[0]

    loss_ref[...] += jnp.reshape(jnp.sum((quant - x) ** 2), (1, 1))


@functools.partial(jax.jit, static_argnames=("tblk",))
def _vq(inputs, embedding_weight, tblk=512):
    B, C, T = inputs.shape
    nt = T // tblk
    embt = jnp.transpose(embedding_weight, (1, 0))

    quant, idx4, loss = pl.pallas_call(
        _vq_body,
        grid=(B, nt),
        in_specs=[
            pl.BlockSpec((1, C, tblk), lambda b, t: (b, 0, t)),
            pl.BlockSpec((_K, _D), lambda b, t: (0, 0)),
            pl.BlockSpec((_D, _K), lambda b, t: (0, 0)),
        ],
        out_specs=[
            pl.BlockSpec((1, C, tblk), lambda b, t: (b, 0, t)),
            pl.BlockSpec((1, 1, 1, tblk), lambda b, t: (b, t, 0, 0)),
            pl.BlockSpec((1, 1), lambda b, t: (0, 0)),
        ],
        out_shape=[
            jax.ShapeDtypeStruct((B, C, T), jnp.float32),
            jax.ShapeDtypeStruct((B, nt, 1, tblk), jnp.int32),
            jax.ShapeDtypeStruct((1, 1), jnp.float32),
        ],
        scratch_shapes=[pltpu.VMEM((1, _K), jnp.float32)],
    )(inputs, embedding_weight, embt)

    indices = idx4.reshape(B, T)
    m = loss[0, 0] / (B * T * C)
    loss = m + _COMMIT * m
    return quant, loss, indices


def kernel(inputs, embedding_weight):
    return _vq(inputs, embedding_weight)


# speedup vs baseline: 1.2239x; 1.2239x over previous
import functools

import jax
import jax.numpy as jnp
from jax.experimental import pallas as pl
from jax.experimental.pallas import tpu as pltpu

_K = 1024
_D = 64
_COMMIT = 0.25


def _vq_body(x_ref, e_ref, et_ref, q_ref, i_ref, loss_ref, esq_ref):
    b = pl.program_id(0)
    tb = pl.program_id(1)
    x = x_ref[0]
    emb = e_ref[...]
    embt = et_ref[...]
    tblk = x.shape[1]

    @pl.when((b == 0) & (tb == 0))
    def _init():
        esq_ref[...] = jnp.sum(emb * emb, axis=1, keepdims=True)
        loss_ref[...] = jnp.zeros((1, 1), jnp.float32)

    x_sq = jnp.sum(x * x, axis=0, keepdims=True)
    e_sq = esq_ref[...]
    xe = jax.lax.dot_general(
        emb, x, (((1,), (0,)), ((), ())),
        preferred_element_type=jnp.float32)
    dist = (x_sq + e_sq) - 2.0 * xe

    minv = jnp.min(dist, axis=0, keepdims=True)
    kiota = jax.lax.broadcasted_iota(jnp.int32, (_K, tblk), 0)
    sel = jnp.where(dist == minv, kiota, _K)
    idx_row = jnp.min(sel, axis=0, keepdims=True)

    onehot_t = (kiota == idx_row).astype(jnp.float32)
    quant = jax.lax.dot_general(
        embt, onehot_t, (((1,), (0,)), ((), ())),
        preferred_element_type=jnp.float32)

    q_ref[0] = x + (quant - x)
    i_ref[0, 0, 0] = idx_row[0]

    loss_ref[...] += jnp.reshape(jnp.sum((quant - x) ** 2), (1, 1))


@functools.partial(jax.jit, static_argnames=("tblk",))
def _vq(inputs, embedding_weight, tblk=512):
    B, C, T = inputs.shape
    nt = T // tblk
    embt = jnp.transpose(embedding_weight, (1, 0))

    quant, idx4, loss = pl.pallas_call(
        _vq_body,
        grid=(B, nt),
        in_specs=[
            pl.BlockSpec((1, C, tblk), lambda b, t: (b, 0, t)),
            pl.BlockSpec((_K, _D), lambda b, t: (0, 0)),
            pl.BlockSpec((_D, _K), lambda b, t: (0, 0)),
        ],
        out_specs=[
            pl.BlockSpec((1, C, tblk), lambda b, t: (b, 0, t)),
            pl.BlockSpec((1, 1, 1, tblk), lambda b, t: (b, t, 0, 0)),
            pl.BlockSpec((1, 1), lambda b, t: (0, 0)),
        ],
        out_shape=[
            jax.ShapeDtypeStruct((B, C, T), jnp.float32),
            jax.ShapeDtypeStruct((B, nt, 1, tblk), jnp.int32),
            jax.ShapeDtypeStruct((1, 1), jnp.float32),
        ],
        scratch_shapes=[pltpu.VMEM((_K, 1), jnp.float32)],
    )(inputs, embedding_weight, embt)

    indices = idx4.reshape(B, T)
    m = loss[0, 0] / (B * T * C)
    loss = m + _COMMIT * m
    return quant, loss, indices


def kernel(inputs, embedding_weight):
    return _vq(inputs, embedding_weight)


# speedup vs baseline: 1.5159x; 1.2385x over previous
import functools

import jax
import jax.numpy as jnp
from jax.experimental import pallas as pl
from jax.experimental.pallas import tpu as pltpu

_K = 1024
_D = 64
_COMMIT = 0.25


def _vq_body(x_ref, e_ref, e2_ref, et_ref, q_ref, i_ref, loss_ref, esq_ref):
    b = pl.program_id(0)
    tb = pl.program_id(1)
    x = x_ref[0]
    emb = e_ref[...]
    emb2 = e2_ref[...]
    embt = et_ref[...]
    tblk = x.shape[1]

    @pl.when((b == 0) & (tb == 0))
    def _init():
        esq_ref[...] = jnp.sum(emb * emb, axis=1, keepdims=True)
        loss_ref[...] = jnp.zeros((1, 1), jnp.float32)

    x_sq = jnp.sum(x * x, axis=0, keepdims=True)
    e_sq = esq_ref[...]
    xe2 = jax.lax.dot_general(
        emb2, x, (((1,), (0,)), ((), ())),
        preferred_element_type=jnp.float32)
    dist = (x_sq + e_sq) - xe2

    minv = jnp.min(dist, axis=0, keepdims=True)
    kiota = jax.lax.broadcasted_iota(jnp.int32, (_K, tblk), 0)
    sel = jnp.where(dist == minv, kiota, _K)
    idx_row = jnp.min(sel, axis=0, keepdims=True)

    onehot_t = (kiota == idx_row).astype(jnp.float32)
    quant = jax.lax.dot_general(
        embt, onehot_t, (((1,), (0,)), ((), ())),
        preferred_element_type=jnp.float32)

    q_ref[0] = x + (quant - x)
    i_ref[0, 0, 0] = idx_row[0]

    loss_ref[...] += jnp.reshape(jnp.sum((quant - x) ** 2), (1, 1))


@functools.partial(jax.jit, static_argnames=("tblk",))
def _vq(inputs, embedding_weight, tblk=1024):
    B, C, T = inputs.shape
    nt = T // tblk
    embt = jnp.transpose(embedding_weight, (1, 0))

    quant, idx4, loss = pl.pallas_call(
        _vq_body,
        grid=(B, nt),
        in_specs=[
            pl.BlockSpec((1, C, tblk), lambda b, t: (b, 0, t)),
            pl.BlockSpec((_K, _D), lambda b, t: (0, 0)),
            pl.BlockSpec((_K, _D), lambda b, t: (0, 0)),
            pl.BlockSpec((_D, _K), lambda b, t: (0, 0)),
        ],
        out_specs=[
            pl.BlockSpec((1, C, tblk), lambda b, t: (b, 0, t)),
            pl.BlockSpec((1, 1, 1, tblk), lambda b, t: (b, t, 0, 0)),
            pl.BlockSpec((1, 1), lambda b, t: (0, 0)),
        ],
        out_shape=[
            jax.ShapeDtypeStruct((B, C, T), jnp.float32),
            jax.ShapeDtypeStruct((B, nt, 1, tblk), jnp.int32),
            jax.ShapeDtypeStruct((1, 1), jnp.float32),
        ],
        scratch_shapes=[pltpu.VMEM((_K, 1), jnp.float32)],
    )(inputs, embedding_weight, embedding_weight + embedding_weight, embt)

    indices = idx4.reshape(B, T)
    m = loss[0, 0] / (B * T * C)
    loss = m + _COMMIT * m
    return quant, loss, indices


def kernel(inputs, embedding_weight):
    return _vq(inputs, embedding_weight)


# speedup vs baseline: 1.6911x; 1.1156x over previous
import functools

import jax
import jax.numpy as jnp
from jax.experimental import pallas as pl
from jax.experimental.pallas import tpu as pltpu

_K = 1024
_D = 64
_COMMIT = 0.25


def _vq_body(x_ref, e_ref, e2_ref, et_ref, q_ref, i_ref, loss_ref, esq_ref):
    b = pl.program_id(0)
    tb = pl.program_id(1)
    x = x_ref[0]
    emb = e_ref[...]
    emb2 = e2_ref[...]
    embt = et_ref[...]
    tblk = x.shape[1]

    @pl.when((b == 0) & (tb == 0))
    def _init():
        esq_ref[...] = jnp.sum(emb * emb, axis=1, keepdims=True)
        loss_ref[...] = jnp.zeros((1, 1), jnp.float32)

    x_sq = jnp.sum(x * x, axis=0, keepdims=True)
    e_sq = esq_ref[...]
    xe2 = jax.lax.dot_general(
        emb2, x, (((1,), (0,)), ((), ())),
        preferred_element_type=jnp.float32)
    dist = (x_sq + e_sq) - xe2

    minv = jnp.min(dist, axis=0, keepdims=True)
    kiota = jax.lax.broadcasted_iota(jnp.int32, (_K, tblk), 0)
    sel = jnp.where(dist == minv, kiota, _K)
    idx_row = jnp.min(sel, axis=0, keepdims=True)

    onehot_t = (kiota == idx_row).astype(jnp.float32)
    quant = jax.lax.dot_general(
        embt, onehot_t, (((1,), (0,)), ((), ())),
        preferred_element_type=jnp.float32)

    q_ref[0] = x + (quant - x)
    i_ref[0, 0, 0] = idx_row[0]

    loss_ref[...] += jnp.reshape(jnp.sum((quant - x) ** 2), (1, 1))


@functools.partial(jax.jit, static_argnames=("tblk",))
def _vq(inputs, embedding_weight, tblk=2048):
    B, C, T = inputs.shape
    nt = T // tblk
    embt = jnp.transpose(embedding_weight, (1, 0))

    quant, idx4, loss = pl.pallas_call(
        _vq_body,
        grid=(B, nt),
        in_specs=[
            pl.BlockSpec((1, C, tblk), lambda b, t: (b, 0, t)),
            pl.BlockSpec((_K, _D), lambda b, t: (0, 0)),
            pl.BlockSpec((_K, _D), lambda b, t: (0, 0)),
            pl.BlockSpec((_D, _K), lambda b, t: (0, 0)),
        ],
        out_specs=[
            pl.BlockSpec((1, C, tblk), lambda b, t: (b, 0, t)),
            pl.BlockSpec((1, 1, 1, tblk), lambda b, t: (b, t, 0, 0)),
            pl.BlockSpec((1, 1), lambda b, t: (0, 0)),
        ],
        out_shape=[
            jax.ShapeDtypeStruct((B, C, T), jnp.float32),
            jax.ShapeDtypeStruct((B, nt, 1, tblk), jnp.int32),
            jax.ShapeDtypeStruct((1, 1), jnp.float32),
        ],
        scratch_shapes=[pltpu.VMEM((_K, 1), jnp.float32)],
    )(inputs, embedding_weight, embedding_weight + embedding_weight, embt)

    indices = idx4.reshape(B, T)
    m = loss[0, 0] / (B * T * C)
    loss = m + _COMMIT * m
    return quant, loss, indices


def kernel(inputs, embedding_weight):
    return _vq(inputs, embedding_weight)


# speedup vs baseline: 1.8220x; 1.0775x over previous
import functools

import jax
import jax.numpy as jnp
from jax.experimental import pallas as pl
from jax.experimental.pallas import tpu as pltpu

_K = 1024
_D = 64
_COMMIT = 0.25


def _vq_body(x_ref, e_ref, e2_ref, et_ref, q_ref, i_ref, loss_ref, esq_ref):
    b = pl.program_id(0)
    tb = pl.program_id(1)
    x = x_ref[0]
    emb = e_ref[...]
    emb2 = e2_ref[...]
    embt = et_ref[...]
    tblk = x.shape[1]

    @pl.when((b == 0) & (tb == 0))
    def _init():
        esq_ref[...] = jnp.sum(emb * emb, axis=1, keepdims=True)
        loss_ref[...] = jnp.zeros((1, 1), jnp.float32)

    x_sq = jnp.sum(x * x, axis=0, keepdims=True)
    e_sq = esq_ref[...]
    xe2 = jax.lax.dot_general(
        emb2, x, (((1,), (0,)), ((), ())),
        preferred_element_type=jnp.float32)
    dist = (x_sq + e_sq) - xe2

    minv = jnp.min(dist, axis=0, keepdims=True)
    kiota = jax.lax.broadcasted_iota(jnp.int32, (_K, tblk), 0)
    sel = jnp.where(dist == minv, kiota, _K)
    idx_row = jnp.min(sel, axis=0, keepdims=True)

    onehot_t = (kiota == idx_row).astype(jnp.float32)
    quant = jax.lax.dot_general(
        embt, onehot_t, (((1,), (0,)), ((), ())),
        preferred_element_type=jnp.float32)

    q_ref[0] = x + (quant - x)
    i_ref[0, 0, 0] = idx_row[0]

    loss_ref[...] += jnp.reshape(jnp.sum((quant - x) ** 2), (1, 1))


@functools.partial(jax.jit, static_argnames=("tblk",))
def _vq(inputs, embedding_weight, tblk=4096):
    B, C, T = inputs.shape
    nt = T // tblk
    embt = jnp.transpose(embedding_weight, (1, 0))

    quant, idx4, loss = pl.pallas_call(
        _vq_body,
        grid=(B, nt),
        in_specs=[
            pl.BlockSpec((1, C, tblk), lambda b, t: (b, 0, t)),
            pl.BlockSpec((_K, _D), lambda b, t: (0, 0)),
            pl.BlockSpec((_K, _D), lambda b, t: (0, 0)),
            pl.BlockSpec((_D, _K), lambda b, t: (0, 0)),
        ],
        out_specs=[
            pl.BlockSpec((1, C, tblk), lambda b, t: (b, 0, t)),
            pl.BlockSpec((1, 1, 1, tblk), lambda b, t: (b, t, 0, 0)),
            pl.BlockSpec((1, 1), lambda b, t: (0, 0)),
        ],
        out_shape=[
            jax.ShapeDtypeStruct((B, C, T), jnp.float32),
            jax.ShapeDtypeStruct((B, nt, 1, tblk), jnp.int32),
            jax.ShapeDtypeStruct((1, 1), jnp.float32),
        ],
        scratch_shapes=[pltpu.VMEM((_K, 1), jnp.float32)],
    )(inputs, embedding_weight, embedding_weight + embedding_weight, embt)

    indices = idx4.reshape(B, T)
    m = loss[0, 0] / (B * T * C)
    loss = m + _COMMIT * m
    return quant, loss, indices


def kernel(inputs, embedding_weight):
    return _vq(inputs, embedding_weight)


# speedup vs baseline: 2.2467x; 1.2331x over previous
import functools

import jax
import jax.numpy as jnp
from jax.experimental import pallas as pl
from jax.experimental.pallas import tpu as pltpu

_K = 1024
_D = 64
_COMMIT = 0.25


def _vq_body(x_ref, e_ref, e2_ref, et_ref, q_ref, i_ref, loss_ref, esq_ref):
    b = pl.program_id(0)
    tb = pl.program_id(1)
    x = x_ref[0]
    emb = e_ref[...]
    emb2 = e2_ref[...]
    embt = et_ref[...]
    tblk = x.shape[1]

    @pl.when((b == 0) & (tb == 0))
    def _init():
        esq_ref[...] = jnp.sum(emb * emb, axis=1, keepdims=True)
        loss_ref[...] = jnp.zeros((1, 1), jnp.float32)

    x_sq = jnp.sum(x * x, axis=0, keepdims=True)
    e_sq = esq_ref[...]
    xe2 = jax.lax.dot_general(
        emb2, x, (((1,), (0,)), ((), ())),
        preferred_element_type=jnp.float32)
    dist = (x_sq + e_sq) - xe2

    idx_row = jnp.argmin(dist, axis=0)[None, :]
    kiota = jax.lax.broadcasted_iota(jnp.int32, (_K, tblk), 0)

    onehot_t = (kiota == idx_row).astype(jnp.float32)
    quant = jax.lax.dot_general(
        embt, onehot_t, (((1,), (0,)), ((), ())),
        preferred_element_type=jnp.float32)

    q_ref[0] = x + (quant - x)
    i_ref[0, 0, 0] = idx_row[0]

    loss_ref[...] += jnp.reshape(jnp.sum((quant - x) ** 2), (1, 1))


@functools.partial(jax.jit, static_argnames=("tblk",))
def _vq(inputs, embedding_weight, tblk=4096):
    B, C, T = inputs.shape
    nt = T // tblk
    embt = jnp.transpose(embedding_weight, (1, 0))

    quant, idx4, loss = pl.pallas_call(
        _vq_body,
        grid=(B, nt),
        in_specs=[
            pl.BlockSpec((1, C, tblk), lambda b, t: (b, 0, t)),
            pl.BlockSpec((_K, _D), lambda b, t: (0, 0)),
            pl.BlockSpec((_K, _D), lambda b, t: (0, 0)),
            pl.BlockSpec((_D, _K), lambda b, t: (0, 0)),
        ],
        out_specs=[
            pl.BlockSpec((1, C, tblk), lambda b, t: (b, 0, t)),
            pl.BlockSpec((1, 1, 1, tblk), lambda b, t: (b, t, 0, 0)),
            pl.BlockSpec((1, 1), lambda b, t: (0, 0)),
        ],
        out_shape=[
            jax.ShapeDtypeStruct((B, C, T), jnp.float32),
            jax.ShapeDtypeStruct((B, nt, 1, tblk), jnp.int32),
            jax.ShapeDtypeStruct((1, 1), jnp.float32),
        ],
        scratch_shapes=[pltpu.VMEM((_K, 1), jnp.float32)],
    )(inputs, embedding_weight, embedding_weight + embedding_weight, embt)

    indices = idx4.reshape(B, T)
    m = loss[0, 0] / (B * T * C)
    loss = m + _COMMIT * m
    return quant, loss, indices


def kernel(inputs, embedding_weight):
    return _vq(inputs, embedding_weight)
